# baseline (speedup 1.0000x reference)
import functools

import jax
import jax.numpy as jnp
from jax import lax
from jax.experimental import pallas as pl
from jax.experimental.pallas import tpu as pltpu
from jax.experimental.pallas import tpu_sc as plsc

N_NODES = 10000
N_EDGES = 320000
IN_CH = 128
HID = 512
OUT = 64

NC = 2
NS = 16
NW = NC * NS
CHUNK = 128
C = 80
EPAD = NW * C * CHUNK
SLAB = 20
NSLAB = C // SLAB
ACC_ROWS = 10240
RPT = ACC_ROWS // NS
DEG_W = 16

_mesh = plsc.VectorSubcoreMesh(core_axis_name="c", subcore_axis_name="s")


def _zero_vmem(buf, rows, width, value=0.0):
    def body(i, _):
        for k in range(width // 16):
            buf[i, pl.ds(k * 16, 16)] = jnp.full((16,), value, jnp.float32)
        return 0
    lax.fori_loop(0, rows, body, 0)


def _make_agg(F):
    @functools.partial(
        pl.kernel,
        mesh=_mesh,
        compiler_params=pltpu.CompilerParams(
            needs_layout_passes=False, use_tc_tiling_on_sc=False),
        out_type=jax.ShapeDtypeStruct((NC, ACC_ROWS, F), jnp.float32),
        scratch_types=[
            pltpu.VMEM((SLAB, CHUNK), jnp.int32),
            pltpu.VMEM((SLAB, CHUNK), jnp.int32),
            pltpu.VMEM((SLAB, CHUNK), jnp.int32),
            pltpu.VMEM((SLAB, CHUNK), jnp.int32),
            pltpu.VMEM((CHUNK, F), jnp.float32),
            pltpu.VMEM((CHUNK, F), jnp.float32),
            pltpu.VMEM_SHARED((ACC_ROWS, F), jnp.float32),
            pltpu.SemaphoreType.DMA,
            pltpu.SemaphoreType.DMA,
            pltpu.SemaphoreType.DMA,
        ],
    )
    def agg(xp_hbm, src_hbm, dst_hbm, out_hbm, ss0, ss1, sd0, sd1,
            rows0, rows1, acc_sh, sem0, sem1, semi):
        cid = lax.axis_index("c")
        sid = lax.axis_index("s")
        wid = sid * NC + cid
        sslabs = (ss0, ss1)
        dslabs = (sd0, sd1)

        _zero_vmem(rows0, CHUNK, F)
        for r in range(RPT // CHUNK):
            pltpu.sync_copy(rows0, acc_sh.at[pl.ds(sid * RPT + r * CHUNK, CHUNK)])
        plsc.subcore_barrier()

        pltpu.sync_copy(src_hbm.at[wid, pl.ds(0, SLAB)], ss0)
        pltpu.sync_copy(dst_hbm.at[wid, pl.ds(0, SLAB)], sd0)
        pltpu.async_copy(xp_hbm.at[ss0.at[0]], rows0, sem0)

        for sl in range(NSLAB):
            ssc, sdc = sslabs[sl % 2], dslabs[sl % 2]
            ssn, sdn = sslabs[(sl + 1) % 2], dslabs[(sl + 1) % 2]
            if sl + 1 < NSLAB:
                pltpu.async_copy(
                    src_hbm.at[wid, pl.ds((sl + 1) * SLAB, SLAB)], ssn, semi)
                pltpu.async_copy(
                    dst_hbm.at[wid, pl.ds((sl + 1) * SLAB, SLAB)], sdn, semi)

            def body(t, _, ssc=ssc, sdc=sdc):
                @pl.when(t % 2 == 0)
                def _even():
                    pltpu.async_copy(xp_hbm.at[ssc.at[t + 1]], rows1, sem1)
                    pltpu.make_async_copy(
                        xp_hbm.at[ssc.at[t]], rows0, sem0).wait()
                    pltpu.sync_copy(rows0, acc_sh.at[sdc.at[t]], add=True)

                @pl.when(t % 2 == 1)
                def _odd():
                    pltpu.async_copy(xp_hbm.at[ssc.at[t + 1]], rows0, sem0)
                    pltpu.make_async_copy(
                        xp_hbm.at[ssc.at[t]], rows1, sem1).wait()
                    pltpu.sync_copy(rows1, acc_sh.at[sdc.at[t]], add=True)

                return 0

            lax.fori_loop(0, SLAB - 1, body, 0)
            if sl + 1 < NSLAB:
                pltpu.make_async_copy(
                    src_hbm.at[wid, pl.ds((sl + 1) * SLAB, SLAB)], ssn,
                    semi).wait()
                pltpu.make_async_copy(
                    dst_hbm.at[wid, pl.ds((sl + 1) * SLAB, SLAB)], sdn,
                    semi).wait()
                pltpu.async_copy(xp_hbm.at[ssn.at[0]], rows0, sem0)
            pltpu.make_async_copy(
                xp_hbm.at[ssc.at[SLAB - 1]], rows1, sem1).wait()
            pltpu.sync_copy(rows1, acc_sh.at[sdc.at[SLAB - 1]], add=True)

        plsc.subcore_barrier()
        pltpu.sync_copy(acc_sh.at[pl.ds(sid * RPT, RPT)],
                        out_hbm.at[cid, pl.ds(sid * RPT, RPT)])

    return agg


@functools.partial(
    pl.kernel,
    mesh=_mesh,
    compiler_params=pltpu.CompilerParams(needs_layout_passes=False),
    out_type=jax.ShapeDtypeStruct((NW, ACC_ROWS // 16, 16), jnp.float32),
    scratch_types=[
        pltpu.VMEM((C, CHUNK), jnp.int32),
        pltpu.VMEM((ACC_ROWS // 16, 16), jnp.float32),
    ],
)
def _deg_kernel(dst_hbm, out_hbm, dst_v, deg_v):
    cid = lax.axis_index("c")
    sid = lax.axis_index("s")
    wid = sid * NC + cid

    pltpu.sync_copy(dst_hbm.at[wid], dst_v)

    def zbody(i, _):
        deg_v[i, :] = jnp.zeros((16,), jnp.float32)
        return 0

    lax.fori_loop(0, ACC_ROWS // 16, zbody, 0)
    ones16 = jnp.full((16,), 1.0, jnp.float32)

    def body(j, _):
        for k in range(CHUNK // 16):
            idx = dst_v[j, pl.ds(k * 16, 16)]
            plsc.addupdate_scatter(deg_v, [idx >> 4, idx & 15], ones16)
        return 0

    lax.fori_loop(0, C, body, 0)
    pltpu.sync_copy(deg_v, out_hbm.at[wid])


_RB = 1024
_GRID = ACC_ROWS // _RB


def _dinv_from_deg(degp_ref):
    i = pl.program_id(0)
    d = jnp.sum(degp_ref[:, pl.ds(i * _RB, _RB)], axis=0)[:, None] + 1.0
    return lax.rsqrt(d)


def _prescale_body(degp_ref, x_ref, o_ref):
    o_ref[...] = _dinv_from_deg(degp_ref) * x_ref[...]


_prescale = pl.pallas_call(
    _prescale_body,
    grid=(_GRID,),
    in_specs=[
        pl.BlockSpec((NW, ACC_ROWS), lambda i: (0, 0)),
        pl.BlockSpec((_RB, IN_CH), lambda i: (i, 0)),
    ],
    out_specs=pl.BlockSpec((_RB, IN_CH), lambda i: (i, 0)),
    out_shape=jax.ShapeDtypeStruct((N_NODES, IN_CH), jnp.float32),
)


def _mid_body(p_ref, xp_ref, degp_ref, w1_ref, b1_ref, a1_ref, w2_ref, o_ref):
    dinv = _dinv_from_deg(degp_ref)
    t = dinv * (p_ref[0] + p_ref[1] + xp_ref[...])
    h = jnp.dot(t, w1_ref[...], preferred_element_type=jnp.float32) + b1_ref[...]
    h = jnp.where(h >= 0, h, a1_ref[...] * h)
    g = jnp.dot(h, w2_ref[...], preferred_element_type=jnp.float32)
    o_ref[...] = dinv * g


_mid = pl.pallas_call(
    _mid_body,
    grid=(_GRID,),
    in_specs=[
        pl.BlockSpec((NC, _RB, IN_CH), lambda i: (0, i, 0)),
        pl.BlockSpec((_RB, IN_CH), lambda i: (i, 0)),
        pl.BlockSpec((NW, ACC_ROWS), lambda i: (0, 0)),
        pl.BlockSpec((IN_CH, HID), lambda i: (0, 0)),
        pl.BlockSpec((1, HID), lambda i: (0, 0)),
        pl.BlockSpec((1, HID), lambda i: (0, 0)),
        pl.BlockSpec((HID, OUT), lambda i: (0, 0)),
    ],
    out_specs=pl.BlockSpec((_RB, OUT), lambda i: (i, 0)),
    out_shape=jax.ShapeDtypeStruct((N_NODES, OUT), jnp.float32),
)


def _final_body(q_ref, gp_ref, degp_ref, b2_ref, a2_ref, o_ref):
    dinv = _dinv_from_deg(degp_ref)
    v = dinv * (q_ref[0] + q_ref[1] + gp_ref[...]) + b2_ref[...]
    o_ref[...] = jnp.where(v >= 0, v, a2_ref[...] * v)


_final = pl.pallas_call(
    _final_body,
    grid=(_GRID,),
    in_specs=[
        pl.BlockSpec((NC, _RB, OUT), lambda i: (0, i, 0)),
        pl.BlockSpec((_RB, OUT), lambda i: (i, 0)),
        pl.BlockSpec((NW, ACC_ROWS), lambda i: (0, 0)),
        pl.BlockSpec((1, OUT), lambda i: (0, 0)),
        pl.BlockSpec((1, OUT), lambda i: (0, 0)),
    ],
    out_specs=pl.BlockSpec((_RB, OUT), lambda i: (i, 0)),
    out_shape=jax.ShapeDtypeStruct((N_NODES, OUT), jnp.float32),
)


_agg128 = _make_agg(IN_CH)
_agg64 = _make_agg(OUT)


def kernel(x, edge_index, W1, b1, a1, W2, b2, a2):
    src = edge_index[0].astype(jnp.int32)
    dst = edge_index[1].astype(jnp.int32)
    pad = EPAD - N_EDGES
    srcp = jnp.concatenate([src, jnp.zeros((pad,), jnp.int32)]).reshape(NW, C, CHUNK)
    dstp = jnp.concatenate([dst, jnp.full((pad,), N_NODES, jnp.int32)]).reshape(NW, C, CHUNK)

    degp = _deg_kernel(dstp).reshape(NW, ACC_ROWS)
    xp = _prescale(degp, x)
    p = _agg128(xp, srcp, dstp)
    gp = _mid(p, xp, degp, W1, b1.reshape(1, HID), a1.reshape(1, HID), W2)
    q = _agg64(gp, srcp, dstp)
    return _final(q, gp, degp, b2.reshape(1, OUT), a2.reshape(1, OUT))


# speedup vs baseline: 3.0489x; 3.0489x over previous
import functools

import jax
import jax.numpy as jnp
from jax import lax
from jax.experimental import pallas as pl
from jax.experimental.pallas import tpu as pltpu
from jax.experimental.pallas import tpu_sc as plsc

N_NODES = 10000
N_EDGES = 320000
IN_CH = 128
HID = 512
OUT = 64

NC = 2
NS = 16
NW = NC * NS
CHUNK = 128
C = 80
EPAD = NW * C * CHUNK
SLAB = 20
NSLAB = C // SLAB
ACC_ROWS = 10240
RPT = ACC_ROWS // NS
DEG_W = 16

_mesh = plsc.VectorSubcoreMesh(core_axis_name="c", subcore_axis_name="s")


def _zero_vmem(buf, rows, width, value=0.0):
    def body(i, _):
        for k in range(width // 16):
            buf[i, pl.ds(k * 16, 16)] = jnp.full((16,), value, jnp.float32)
        return 0
    lax.fori_loop(0, rows, body, 0)


def _make_agg(F, DEPTH):
    assert SLAB % DEPTH == 0

    @functools.partial(
        pl.kernel,
        mesh=_mesh,
        compiler_params=pltpu.CompilerParams(
            needs_layout_passes=False, use_tc_tiling_on_sc=False),
        out_type=jax.ShapeDtypeStruct((NC, ACC_ROWS, F), jnp.float32),
        scratch_types=[
            pltpu.VMEM((SLAB, CHUNK), jnp.int32),
            pltpu.VMEM((SLAB, CHUNK), jnp.int32),
            pltpu.VMEM((SLAB, CHUNK), jnp.int32),
            pltpu.VMEM((SLAB, CHUNK), jnp.int32),
        ] + [pltpu.VMEM((CHUNK, F), jnp.float32) for _ in range(DEPTH)]
          + [pltpu.VMEM_SHARED((ACC_ROWS, F), jnp.float32)]
          + [pltpu.SemaphoreType.DMA for _ in range(DEPTH + 1)],
    )
    def agg(xp_hbm, ei_hbm, out_hbm, ss0, ss1, sd0, sd1, *scr):
        rows = scr[:DEPTH]
        acc_sh = scr[DEPTH]
        sems = scr[DEPTH + 1:DEPTH + 1 + DEPTH]
        semi = scr[DEPTH + 1 + DEPTH]
        cid = lax.axis_index("c")
        sid = lax.axis_index("s")
        wid = sid * NC + cid
        sslabs = (ss0, ss1)
        dslabs = (sd0, sd1)

        pltpu.sync_copy(ei_hbm.at[0, wid, pl.ds(0, SLAB)], ss0)
        pltpu.sync_copy(ei_hbm.at[1, wid, pl.ds(0, SLAB)], sd0)
        for k in range(1, DEPTH - 1):
            pltpu.async_copy(xp_hbm.at[ss0.at[k]], rows[k], sems[k])

        _zero_vmem(rows[0], CHUNK, F)
        for r in range(RPT // CHUNK):
            pltpu.sync_copy(rows[0],
                            acc_sh.at[pl.ds(sid * RPT + r * CHUNK, CHUNK)])
        pltpu.async_copy(xp_hbm.at[ss0.at[0]], rows[0], sems[0])
        plsc.subcore_barrier()

        for sl in range(NSLAB):
            ssc, sdc = sslabs[sl % 2], dslabs[sl % 2]
            ssn, sdn = sslabs[(sl + 1) % 2], dslabs[(sl + 1) % 2]
            if sl + 1 < NSLAB:
                pltpu.async_copy(
                    ei_hbm.at[0, wid, pl.ds((sl + 1) * SLAB, SLAB)], ssn, semi)
                pltpu.async_copy(
                    ei_hbm.at[1, wid, pl.ds((sl + 1) * SLAB, SLAB)], sdn, semi)

            def body(t, _, ssc=ssc, sdc=sdc):
                for r in range(DEPTH):
                    @pl.when(t % DEPTH == r)
                    def _work(r=r):
                        bn = (r + DEPTH - 1) % DEPTH
                        pltpu.async_copy(
                            xp_hbm.at[ssc.at[t + DEPTH - 1]], rows[bn],
                            sems[bn])
                        pltpu.make_async_copy(
                            xp_hbm.at[ssc.at[t]], rows[r], sems[r]).wait()
                        pltpu.sync_copy(
                            rows[r], acc_sh.at[sdc.at[t]], add=True)

                return 0

            lax.fori_loop(0, SLAB - (DEPTH - 1), body, 0)
            if sl + 1 < NSLAB:
                pltpu.make_async_copy(
                    ei_hbm.at[0, wid, pl.ds((sl + 1) * SLAB, SLAB)], ssn,
                    semi).wait()
                pltpu.make_async_copy(
                    ei_hbm.at[1, wid, pl.ds((sl + 1) * SLAB, SLAB)], sdn,
                    semi).wait()
            for e in range(DEPTH - 1):
                t = SLAB - (DEPTH - 1) + e
                b = t % DEPTH
                bn = (t + DEPTH - 1) % DEPTH
                if sl + 1 < NSLAB:
                    pltpu.async_copy(
                        xp_hbm.at[ssn.at[e]], rows[bn], sems[bn])
                pltpu.make_async_copy(
                    xp_hbm.at[ssc.at[t]], rows[b], sems[b]).wait()
                pltpu.sync_copy(rows[b], acc_sh.at[sdc.at[t]], add=True)

        plsc.subcore_barrier()
        pltpu.sync_copy(acc_sh.at[pl.ds(sid * RPT, RPT)],
                        out_hbm.at[cid, pl.ds(sid * RPT, RPT)])

    return agg


@functools.partial(
    pl.kernel,
    mesh=_mesh,
    compiler_params=pltpu.CompilerParams(
        needs_layout_passes=False, use_tc_tiling_on_sc=False),
    out_type=jax.ShapeDtypeStruct((NC, ACC_ROWS), jnp.float32),
    scratch_types=[
        pltpu.VMEM((C, CHUNK), jnp.int32),
        pltpu.VMEM((ACC_ROWS,), jnp.float32),
        pltpu.VMEM((NS, RPT), jnp.float32),
        pltpu.VMEM_SHARED((NS, ACC_ROWS), jnp.float32),
    ],
)
def _deg_kernel(ei_hbm, out_hbm, dst_v, deg_v, part_v, sh):
    cid = lax.axis_index("c")
    sid = lax.axis_index("s")
    wid = sid * NC + cid

    pltpu.sync_copy(ei_hbm.at[1, wid], dst_v)

    def zbody(i, _):
        for u in range(4):
            deg_v[pl.ds(i * 64 + u * 16, 16)] = jnp.zeros((16,), jnp.float32)
        return 0

    lax.fori_loop(0, ACC_ROWS // 64, zbody, 0)
    ones16 = jnp.full((16,), 1.0, jnp.float32)

    def body(j, _):
        for k in range(CHUNK // 16):
            idx = dst_v[j, pl.ds(k * 16, 16)]
            plsc.addupdate_scatter(deg_v, [idx], ones16)
        return 0

    lax.fori_loop(0, C, body, 0)

    pltpu.sync_copy(deg_v, sh.at[sid])
    plsc.subcore_barrier()
    pltpu.sync_copy(sh.at[:, pl.ds(sid * RPT, RPT)], part_v)

    def rbody(i, _):
        acc = part_v[0, pl.ds(i * 16, 16)]
        for t in range(1, NS):
            acc = acc + part_v[t, pl.ds(i * 16, 16)]
        deg_v[pl.ds(i * 16, 16)] = acc
        return 0

    lax.fori_loop(0, RPT // 16, rbody, 0)
    pltpu.sync_copy(deg_v.at[pl.ds(0, RPT)],
                    out_hbm.at[cid, pl.ds(sid * RPT, RPT)])


_RB = 1024
_GRID = ACC_ROWS // _RB


def _dinv_from_deg(degp_ref):
    i = pl.program_id(0)
    sl = pl.ds(i * _RB, _RB)
    d = (degp_ref[0, sl] + degp_ref[1, sl])[:, None] + 1.0
    return lax.rsqrt(d)


def _prescale_body(degp_ref, x_ref, o_ref):
    o_ref[...] = _dinv_from_deg(degp_ref) * x_ref[...]


_prescale = pl.pallas_call(
    _prescale_body,
    grid=(_GRID,),
    in_specs=[
        pl.BlockSpec((NC, ACC_ROWS), lambda i: (0, 0)),
        pl.BlockSpec((_RB, IN_CH), lambda i: (i, 0)),
    ],
    out_specs=pl.BlockSpec((_RB, IN_CH), lambda i: (i, 0)),
    out_shape=jax.ShapeDtypeStruct((N_NODES, IN_CH), jnp.float32),
)


def _mid_body(p_ref, xp_ref, degp_ref, w1_ref, b1_ref, a1_ref, w2_ref, o_ref):
    dinv = _dinv_from_deg(degp_ref)
    t = (dinv * (p_ref[0] + p_ref[1] + xp_ref[...])).astype(jnp.bfloat16)
    h = jnp.dot(t, w1_ref[...].astype(jnp.bfloat16),
                preferred_element_type=jnp.float32) + b1_ref[...]
    h = jnp.where(h >= 0, h, a1_ref[...] * h).astype(jnp.bfloat16)
    g = jnp.dot(h, w2_ref[...].astype(jnp.bfloat16),
                preferred_element_type=jnp.float32)
    o_ref[...] = dinv * g


_mid = pl.pallas_call(
    _mid_body,
    grid=(_GRID,),
    in_specs=[
        pl.BlockSpec((NC, _RB, IN_CH), lambda i: (0, i, 0)),
        pl.BlockSpec((_RB, IN_CH), lambda i: (i, 0)),
        pl.BlockSpec((NC, ACC_ROWS), lambda i: (0, 0)),
        pl.BlockSpec((IN_CH, HID), lambda i: (0, 0)),
        pl.BlockSpec((1, HID), lambda i: (0, 0)),
        pl.BlockSpec((1, HID), lambda i: (0, 0)),
        pl.BlockSpec((HID, OUT), lambda i: (0, 0)),
    ],
    out_specs=pl.BlockSpec((_RB, OUT), lambda i: (i, 0)),
    out_shape=jax.ShapeDtypeStruct((N_NODES, OUT), jnp.float32),
)


def _final_body(q_ref, gp_ref, degp_ref, b2_ref, a2_ref, o_ref):
    dinv = _dinv_from_deg(degp_ref)
    v = dinv * (q_ref[0] + q_ref[1] + gp_ref[...]) + b2_ref[...]
    o_ref[...] = jnp.where(v >= 0, v, a2_ref[...] * v)


_final = pl.pallas_call(
    _final_body,
    grid=(_GRID,),
    in_specs=[
        pl.BlockSpec((NC, _RB, OUT), lambda i: (0, i, 0)),
        pl.BlockSpec((_RB, OUT), lambda i: (i, 0)),
        pl.BlockSpec((NC, ACC_ROWS), lambda i: (0, 0)),
        pl.BlockSpec((1, OUT), lambda i: (0, 0)),
        pl.BlockSpec((1, OUT), lambda i: (0, 0)),
    ],
    out_specs=pl.BlockSpec((_RB, OUT), lambda i: (i, 0)),
    out_shape=jax.ShapeDtypeStruct((N_NODES, OUT), jnp.float32),
)


_agg128 = _make_agg(IN_CH, 2)
_agg64 = _make_agg(OUT, 4)


def kernel(x, edge_index, W1, b1, a1, W2, b2, a2):
    pad = EPAD - N_EDGES
    pad_src = jnp.arange(pad, dtype=jnp.int32) % N_NODES
    pad_dst = N_NODES + jnp.arange(pad, dtype=jnp.int32) % (ACC_ROWS - N_NODES)
    ei = jnp.concatenate(
        [edge_index.astype(jnp.int32), jnp.stack([pad_src, pad_dst])], axis=1
    ).reshape(2, NW, C, CHUNK)

    degp = _deg_kernel(ei)
    xp = _prescale(degp, x)
    p = _agg128(xp, ei)
    gp = _mid(p, xp, degp, W1, b1.reshape(1, HID), a1.reshape(1, HID), W2)
    q = _agg64(gp, ei)
    return _final(q, gp, degp, b2.reshape(1, OUT), a2.reshape(1, OUT))


# speedup vs baseline: 3.0518x; 1.0009x over previous
import functools

import jax
import jax.numpy as jnp
from jax import lax
from jax.experimental import pallas as pl
from jax.experimental.pallas import tpu as pltpu
from jax.experimental.pallas import tpu_sc as plsc

N_NODES = 10000
N_EDGES = 320000
IN_CH = 128
HID = 512
OUT = 64

NC = 2
NS = 16
NW = NC * NS
CHUNK = 128
C = 80
EPAD = NW * C * CHUNK
SLAB = 20
NSLAB = C // SLAB
ACC_ROWS = 10240
RPT = ACC_ROWS // NS

_mesh = plsc.VectorSubcoreMesh(core_axis_name="c", subcore_axis_name="s")


def _zero_vmem(buf, rows, width, value=0.0):
    def body(i, _):
        for k in range(width // 16):
            buf[i, pl.ds(k * 16, 16)] = jnp.full((16,), value, jnp.float32)
        return 0
    lax.fori_loop(0, rows, body, 0)


def _make_agg(F, DEPTH):
    assert SLAB % DEPTH == 0

    @functools.partial(
        pl.kernel,
        mesh=_mesh,
        compiler_params=pltpu.CompilerParams(
            needs_layout_passes=False, use_tc_tiling_on_sc=False),
        out_type=jax.ShapeDtypeStruct((NC, ACC_ROWS, F), jnp.float32),
        scratch_types=[
            pltpu.VMEM((SLAB, CHUNK), jnp.int32),
            pltpu.VMEM((SLAB, CHUNK), jnp.int32),
            pltpu.VMEM((SLAB, CHUNK), jnp.int32),
            pltpu.VMEM((SLAB, CHUNK), jnp.int32),
        ] + [pltpu.VMEM((CHUNK, F), jnp.float32) for _ in range(DEPTH)]
          + [pltpu.VMEM_SHARED((ACC_ROWS, F), jnp.float32)]
          + [pltpu.SemaphoreType.DMA for _ in range(DEPTH + 1)],
    )
    def agg(xp_hbm, ei_hbm, out_hbm, ss0, ss1, sd0, sd1, *scr):
        rows = scr[:DEPTH]
        acc_sh = scr[DEPTH]
        sems = scr[DEPTH + 1:DEPTH + 1 + DEPTH]
        semi = scr[DEPTH + 1 + DEPTH]
        cid = lax.axis_index("c")
        sid = lax.axis_index("s")
        wid = sid * NC + cid
        sslabs = (ss0, ss1)
        dslabs = (sd0, sd1)

        pltpu.sync_copy(ei_hbm.at[0, wid, pl.ds(0, SLAB)], ss0)
        pltpu.sync_copy(ei_hbm.at[1, wid, pl.ds(0, SLAB)], sd0)
        for k in range(1, DEPTH - 1):
            pltpu.async_copy(xp_hbm.at[ss0.at[k]], rows[k], sems[k])

        _zero_vmem(rows[0], CHUNK, F)
        for r in range(RPT // CHUNK):
            pltpu.sync_copy(rows[0],
                            acc_sh.at[pl.ds(sid * RPT + r * CHUNK, CHUNK)])
        pltpu.async_copy(xp_hbm.at[ss0.at[0]], rows[0], sems[0])
        plsc.subcore_barrier()

        for sl in range(NSLAB):
            ssc, sdc = sslabs[sl % 2], dslabs[sl % 2]
            ssn, sdn = sslabs[(sl + 1) % 2], dslabs[(sl + 1) % 2]
            if sl + 1 < NSLAB:
                pltpu.async_copy(
                    ei_hbm.at[0, wid, pl.ds((sl + 1) * SLAB, SLAB)], ssn, semi)
                pltpu.async_copy(
                    ei_hbm.at[1, wid, pl.ds((sl + 1) * SLAB, SLAB)], sdn, semi)

            def body(t, _, ssc=ssc, sdc=sdc):
                for r in range(DEPTH):
                    @pl.when(t % DEPTH == r)
                    def _work(r=r):
                        bn = (r + DEPTH - 1) % DEPTH
                        pltpu.async_copy(
                            xp_hbm.at[ssc.at[t + DEPTH - 1]], rows[bn],
                            sems[bn])
                        pltpu.make_async_copy(
                            xp_hbm.at[ssc.at[t]], rows[r], sems[r]).wait()
                        pltpu.sync_copy(
                            rows[r], acc_sh.at[sdc.at[t]], add=True)

                return 0

            lax.fori_loop(0, SLAB - (DEPTH - 1), body, 0)
            if sl + 1 < NSLAB:
                pltpu.make_async_copy(
                    ei_hbm.at[0, wid, pl.ds((sl + 1) * SLAB, SLAB)], ssn,
                    semi).wait()
                pltpu.make_async_copy(
                    ei_hbm.at[1, wid, pl.ds((sl + 1) * SLAB, SLAB)], sdn,
                    semi).wait()
            for e in range(DEPTH - 1):
                t = SLAB - (DEPTH - 1) + e
                b = t % DEPTH
                bn = (t + DEPTH - 1) % DEPTH
                if sl + 1 < NSLAB:
                    pltpu.async_copy(
                        xp_hbm.at[ssn.at[e]], rows[bn], sems[bn])
                pltpu.make_async_copy(
                    xp_hbm.at[ssc.at[t]], rows[b], sems[b]).wait()
                pltpu.sync_copy(rows[b], acc_sh.at[sdc.at[t]], add=True)

        plsc.subcore_barrier()
        pltpu.sync_copy(acc_sh.at[pl.ds(sid * RPT, RPT)],
                        out_hbm.at[cid, pl.ds(sid * RPT, RPT)])

    return agg


@functools.partial(
    pl.kernel,
    mesh=_mesh,
    compiler_params=pltpu.CompilerParams(
        needs_layout_passes=False, use_tc_tiling_on_sc=False),
    out_type=jax.ShapeDtypeStruct((NC, ACC_ROWS), jnp.float32),
    scratch_types=[
        pltpu.VMEM((C, CHUNK), jnp.int32),
        pltpu.VMEM((ACC_ROWS,), jnp.float32),
        pltpu.VMEM((NS, RPT), jnp.float32),
        pltpu.VMEM_SHARED((NS, ACC_ROWS), jnp.float32),
    ],
)
def _deg_kernel(ei_hbm, out_hbm, dst_v, deg_v, part_v, sh):
    cid = lax.axis_index("c")
    sid = lax.axis_index("s")
    wid = sid * NC + cid

    pltpu.sync_copy(ei_hbm.at[1, wid], dst_v)

    def zbody(i, _):
        for u in range(4):
            deg_v[pl.ds(i * 64 + u * 16, 16)] = jnp.zeros((16,), jnp.float32)
        return 0

    lax.fori_loop(0, ACC_ROWS // 64, zbody, 0)
    ones16 = jnp.full((16,), 1.0, jnp.float32)

    def body(j, _):
        for k in range(CHUNK // 16):
            idx = dst_v[j, pl.ds(k * 16, 16)]
            plsc.addupdate_scatter(deg_v, [idx], ones16)
        return 0

    lax.fori_loop(0, C, body, 0)

    pltpu.sync_copy(deg_v, sh.at[sid])
    plsc.subcore_barrier()
    pltpu.sync_copy(sh.at[:, pl.ds(sid * RPT, RPT)], part_v)

    def rbody(i, _):
        acc = part_v[0, pl.ds(i * 16, 16)]
        for t in range(1, NS):
            acc = acc + part_v[t, pl.ds(i * 16, 16)]
        deg_v[pl.ds(i * 16, 16)] = acc
        return 0

    lax.fori_loop(0, RPT // 16, rbody, 0)
    pltpu.sync_copy(deg_v.at[pl.ds(0, RPT)],
                    out_hbm.at[cid, pl.ds(sid * RPT, RPT)])


_RB = 1024
_GRID = ACC_ROWS // _RB


def _dinv_from_deg(degp_ref):
    i = pl.program_id(0)
    sl = pl.ds(i * _RB, _RB)
    d = (degp_ref[0, sl] + degp_ref[1, sl])[:, None] + 1.0
    return lax.rsqrt(d)


def _prescale_body(degp_ref, x_ref, o_ref):
    o_ref[...] = _dinv_from_deg(degp_ref) * x_ref[...]


_prescale = pl.pallas_call(
    _prescale_body,
    grid=(_GRID,),
    in_specs=[
        pl.BlockSpec((NC, ACC_ROWS), lambda i: (0, 0)),
        pl.BlockSpec((_RB, IN_CH), lambda i: (i, 0)),
    ],
    out_specs=pl.BlockSpec((_RB, IN_CH), lambda i: (i, 0)),
    out_shape=jax.ShapeDtypeStruct((N_NODES, IN_CH), jnp.float32),
)


def _mid_body(p_ref, xp_ref, degp_ref, w1_ref, b1_ref, a1_ref, w2_ref, o_ref):
    dinv = _dinv_from_deg(degp_ref)
    t = (dinv * (p_ref[0] + p_ref[1] + xp_ref[...])).astype(jnp.bfloat16)
    h = jnp.dot(t, w1_ref[...].astype(jnp.bfloat16),
                preferred_element_type=jnp.float32) + b1_ref[...]
    h = jnp.where(h >= 0, h, a1_ref[...] * h).astype(jnp.bfloat16)
    g = jnp.dot(h, w2_ref[...].astype(jnp.bfloat16),
                preferred_element_type=jnp.float32)
    o_ref[...] = dinv * g


_mid = pl.pallas_call(
    _mid_body,
    grid=(_GRID,),
    in_specs=[
        pl.BlockSpec((NC, _RB, IN_CH), lambda i: (0, i, 0)),
        pl.BlockSpec((_RB, IN_CH), lambda i: (i, 0)),
        pl.BlockSpec((NC, ACC_ROWS), lambda i: (0, 0)),
        pl.BlockSpec((IN_CH, HID), lambda i: (0, 0)),
        pl.BlockSpec((1, HID), lambda i: (0, 0)),
        pl.BlockSpec((1, HID), lambda i: (0, 0)),
        pl.BlockSpec((HID, OUT), lambda i: (0, 0)),
    ],
    out_specs=pl.BlockSpec((_RB, OUT), lambda i: (i, 0)),
    out_shape=jax.ShapeDtypeStruct((N_NODES, OUT), jnp.float32),
)


def _final_body(q_ref, gp_ref, degp_ref, b2_ref, a2_ref, o_ref):
    dinv = _dinv_from_deg(degp_ref)
    v = dinv * (q_ref[0] + q_ref[1] + gp_ref[...]) + b2_ref[...]
    o_ref[...] = jnp.where(v >= 0, v, a2_ref[...] * v)


_final = pl.pallas_call(
    _final_body,
    grid=(_GRID,),
    in_specs=[
        pl.BlockSpec((NC, _RB, OUT), lambda i: (0, i, 0)),
        pl.BlockSpec((_RB, OUT), lambda i: (i, 0)),
        pl.BlockSpec((NC, ACC_ROWS), lambda i: (0, 0)),
        pl.BlockSpec((1, OUT), lambda i: (0, 0)),
        pl.BlockSpec((1, OUT), lambda i: (0, 0)),
    ],
    out_specs=pl.BlockSpec((_RB, OUT), lambda i: (i, 0)),
    out_shape=jax.ShapeDtypeStruct((N_NODES, OUT), jnp.float32),
)


_agg128 = _make_agg(IN_CH, 2)
_agg64 = _make_agg(OUT, 4)


def kernel(x, edge_index, W1, b1, a1, W2, b2, a2):
    pad = EPAD - N_EDGES
    pad_src = jnp.arange(pad, dtype=jnp.int32) % N_NODES
    pad_dst = N_NODES + jnp.arange(pad, dtype=jnp.int32) % (ACC_ROWS - N_NODES)
    ei = jnp.concatenate(
        [edge_index.astype(jnp.int32), jnp.stack([pad_src, pad_dst])], axis=1
    ).reshape(2, NW, C, CHUNK)

    degp = _deg_kernel(ei)
    xp = _prescale(degp, x)
    p = _agg128(xp, ei)
    gp = _mid(p, xp, degp, W1, b1.reshape(1, HID), a1.reshape(1, HID), W2)
    q = _agg64(gp, ei)
    return _final(q, gp, degp, b2.reshape(1, OUT), a2.reshape(1, OUT))


# speedup vs baseline: 3.0580x; 1.0020x over previous
import functools

import jax
import jax.numpy as jnp
from jax import lax
from jax.experimental import pallas as pl
from jax.experimental.pallas import tpu as pltpu
from jax.experimental.pallas import tpu_sc as plsc

N_NODES = 10000
N_EDGES = 320000
IN_CH = 128
HID = 512
OUT = 64

NC = 2
NS = 16
NW = NC * NS
CHUNK = 128
C = 80
EPAD = NW * C * CHUNK
ACC_ROWS = 10240
RPT = ACC_ROWS // NS

_mesh = plsc.VectorSubcoreMesh(core_axis_name="c", subcore_axis_name="s")


def _zero_vmem(buf, rows, width, value=0.0):
    def body(i, _):
        for k in range(width // 16):
            buf[i, pl.ds(k * 16, 16)] = jnp.full((16,), value, jnp.float32)
        return 0
    lax.fori_loop(0, rows, body, 0)


def _make_agg(F, DEPTH, SLAB):
    assert C % SLAB == 0 and SLAB % DEPTH == 0
    NSLAB = C // SLAB

    @functools.partial(
        pl.kernel,
        mesh=_mesh,
        compiler_params=pltpu.CompilerParams(
            needs_layout_passes=False, use_tc_tiling_on_sc=False),
        out_type=jax.ShapeDtypeStruct((NC, ACC_ROWS, F), jnp.float32),
        scratch_types=[
            pltpu.VMEM((SLAB, CHUNK), jnp.int32),
            pltpu.VMEM((SLAB, CHUNK), jnp.int32),
            pltpu.VMEM((SLAB, CHUNK), jnp.int32),
            pltpu.VMEM((SLAB, CHUNK), jnp.int32),
        ] + [pltpu.VMEM((CHUNK, F), jnp.float32) for _ in range(DEPTH)]
          + [pltpu.VMEM_SHARED((ACC_ROWS, F), jnp.float32)]
          + [pltpu.SemaphoreType.DMA for _ in range(DEPTH + 1)],
    )
    def agg(xp_hbm, ei_hbm, out_hbm, ss0, ss1, sd0, sd1, *scr):
        rows = scr[:DEPTH]
        acc_sh = scr[DEPTH]
        sems = scr[DEPTH + 1:DEPTH + 1 + DEPTH]
        semi = scr[DEPTH + 1 + DEPTH]
        cid = lax.axis_index("c")
        sid = lax.axis_index("s")
        wid = sid * NC + cid
        sslabs = (ss0, ss1)
        dslabs = (sd0, sd1)

        pltpu.sync_copy(ei_hbm.at[0, wid, pl.ds(0, SLAB)], ss0)
        pltpu.sync_copy(ei_hbm.at[1, wid, pl.ds(0, SLAB)], sd0)
        for k in range(1, DEPTH - 1):
            pltpu.async_copy(xp_hbm.at[ss0.at[k]], rows[k], sems[k])

        _zero_vmem(rows[0], CHUNK, F)
        for r in range(RPT // CHUNK):
            pltpu.sync_copy(rows[0],
                            acc_sh.at[pl.ds(sid * RPT + r * CHUNK, CHUNK)])
        pltpu.async_copy(xp_hbm.at[ss0.at[0]], rows[0], sems[0])
        plsc.subcore_barrier()

        for sl in range(NSLAB):
            ssc, sdc = sslabs[sl % 2], dslabs[sl % 2]
            ssn, sdn = sslabs[(sl + 1) % 2], dslabs[(sl + 1) % 2]
            if sl + 1 < NSLAB:
                pltpu.async_copy(
                    ei_hbm.at[0, wid, pl.ds((sl + 1) * SLAB, SLAB)], ssn, semi)
                pltpu.async_copy(
                    ei_hbm.at[1, wid, pl.ds((sl + 1) * SLAB, SLAB)], sdn, semi)

            def body(t, _, ssc=ssc, sdc=sdc):
                for r in range(DEPTH):
                    @pl.when(t % DEPTH == r)
                    def _work(r=r):
                        bn = (r + DEPTH - 1) % DEPTH
                        pltpu.async_copy(
                            xp_hbm.at[ssc.at[t + DEPTH - 1]], rows[bn],
                            sems[bn])
                        pltpu.make_async_copy(
                            xp_hbm.at[ssc.at[t]], rows[r], sems[r]).wait()
                        pltpu.sync_copy(
                            rows[r], acc_sh.at[sdc.at[t]], add=True)

                return 0

            lax.fori_loop(0, SLAB - (DEPTH - 1), body, 0)
            if sl + 1 < NSLAB:
                pltpu.make_async_copy(
                    ei_hbm.at[0, wid, pl.ds((sl + 1) * SLAB, SLAB)], ssn,
                    semi).wait()
                pltpu.make_async_copy(
                    ei_hbm.at[1, wid, pl.ds((sl + 1) * SLAB, SLAB)], sdn,
                    semi).wait()
            for e in range(DEPTH - 1):
                t = SLAB - (DEPTH - 1) + e
                b = t % DEPTH
                bn = (t + DEPTH - 1) % DEPTH
                if sl + 1 < NSLAB:
                    pltpu.async_copy(
                        xp_hbm.at[ssn.at[e]], rows[bn], sems[bn])
                pltpu.make_async_copy(
                    xp_hbm.at[ssc.at[t]], rows[b], sems[b]).wait()
                pltpu.sync_copy(rows[b], acc_sh.at[sdc.at[t]], add=True)

        plsc.subcore_barrier()
        pltpu.sync_copy(acc_sh.at[pl.ds(sid * RPT, RPT)],
                        out_hbm.at[cid, pl.ds(sid * RPT, RPT)])

    return agg


@functools.partial(
    pl.kernel,
    mesh=_mesh,
    compiler_params=pltpu.CompilerParams(
        needs_layout_passes=False, use_tc_tiling_on_sc=False),
    out_type=jax.ShapeDtypeStruct((NC, ACC_ROWS), jnp.float32),
    scratch_types=[
        pltpu.VMEM((C, CHUNK), jnp.int32),
        pltpu.VMEM((ACC_ROWS,), jnp.float32),
        pltpu.VMEM((NS, RPT), jnp.float32),
        pltpu.VMEM_SHARED((NS, ACC_ROWS), jnp.float32),
    ],
)
def _deg_kernel(ei_hbm, out_hbm, dst_v, deg_v, part_v, sh):
    cid = lax.axis_index("c")
    sid = lax.axis_index("s")
    wid = sid * NC + cid

    pltpu.sync_copy(ei_hbm.at[1, wid], dst_v)

    def zbody(i, _):
        for u in range(4):
            deg_v[pl.ds(i * 64 + u * 16, 16)] = jnp.zeros((16,), jnp.float32)
        return 0

    lax.fori_loop(0, ACC_ROWS // 64, zbody, 0)
    ones16 = jnp.full((16,), 1.0, jnp.float32)

    def body(j, _):
        for k in range(CHUNK // 16):
            idx = dst_v[j, pl.ds(k * 16, 16)]
            plsc.addupdate_scatter(deg_v, [idx], ones16)
        return 0

    lax.fori_loop(0, C, body, 0)

    pltpu.sync_copy(deg_v, sh.at[sid])
    plsc.subcore_barrier()
    pltpu.sync_copy(sh.at[:, pl.ds(sid * RPT, RPT)], part_v)

    def rbody(i, _):
        acc = part_v[0, pl.ds(i * 16, 16)]
        for t in range(1, NS):
            acc = acc + part_v[t, pl.ds(i * 16, 16)]
        deg_v[pl.ds(i * 16, 16)] = acc
        return 0

    lax.fori_loop(0, RPT // 16, rbody, 0)
    pltpu.sync_copy(deg_v.at[pl.ds(0, RPT)],
                    out_hbm.at[cid, pl.ds(sid * RPT, RPT)])


_RB = 1024
_GRID = ACC_ROWS // _RB


def _dinv_from_deg(degp_ref):
    i = pl.program_id(0)
    sl = pl.ds(i * _RB, _RB)
    d = (degp_ref[0, sl] + degp_ref[1, sl])[:, None] + 1.0
    return lax.rsqrt(d)


def _prescale_body(degp_ref, x_ref, o_ref):
    o_ref[...] = _dinv_from_deg(degp_ref) * x_ref[...]


_prescale = pl.pallas_call(
    _prescale_body,
    grid=(_GRID,),
    in_specs=[
        pl.BlockSpec((NC, ACC_ROWS), lambda i: (0, 0)),
        pl.BlockSpec((_RB, IN_CH), lambda i: (i, 0)),
    ],
    out_specs=pl.BlockSpec((_RB, IN_CH), lambda i: (i, 0)),
    out_shape=jax.ShapeDtypeStruct((N_NODES, IN_CH), jnp.float32),
)


def _mid_body(p_ref, xp_ref, degp_ref, w1_ref, b1_ref, a1_ref, w2_ref, o_ref):
    dinv = _dinv_from_deg(degp_ref)
    t = (dinv * (p_ref[0] + p_ref[1] + xp_ref[...])).astype(jnp.bfloat16)
    h = jnp.dot(t, w1_ref[...].astype(jnp.bfloat16),
                preferred_element_type=jnp.float32) + b1_ref[...]
    h = jnp.where(h >= 0, h, a1_ref[...] * h).astype(jnp.bfloat16)
    g = jnp.dot(h, w2_ref[...].astype(jnp.bfloat16),
                preferred_element_type=jnp.float32)
    o_ref[...] = dinv * g


_mid = pl.pallas_call(
    _mid_body,
    grid=(_GRID,),
    in_specs=[
        pl.BlockSpec((NC, _RB, IN_CH), lambda i: (0, i, 0)),
        pl.BlockSpec((_RB, IN_CH), lambda i: (i, 0)),
        pl.BlockSpec((NC, ACC_ROWS), lambda i: (0, 0)),
        pl.BlockSpec((IN_CH, HID), lambda i: (0, 0)),
        pl.BlockSpec((1, HID), lambda i: (0, 0)),
        pl.BlockSpec((1, HID), lambda i: (0, 0)),
        pl.BlockSpec((HID, OUT), lambda i: (0, 0)),
    ],
    out_specs=pl.BlockSpec((_RB, OUT), lambda i: (i, 0)),
    out_shape=jax.ShapeDtypeStruct((N_NODES, OUT), jnp.float32),
)


def _final_body(q_ref, gp_ref, degp_ref, b2_ref, a2_ref, o_ref):
    dinv = _dinv_from_deg(degp_ref)
    v = dinv * (q_ref[0] + q_ref[1] + gp_ref[...]) + b2_ref[...]
    o_ref[...] = jnp.where(v >= 0, v, a2_ref[...] * v)


_final = pl.pallas_call(
    _final_body,
    grid=(_GRID,),
    in_specs=[
        pl.BlockSpec((NC, _RB, OUT), lambda i: (0, i, 0)),
        pl.BlockSpec((_RB, OUT), lambda i: (i, 0)),
        pl.BlockSpec((NC, ACC_ROWS), lambda i: (0, 0)),
        pl.BlockSpec((1, OUT), lambda i: (0, 0)),
        pl.BlockSpec((1, OUT), lambda i: (0, 0)),
    ],
    out_specs=pl.BlockSpec((_RB, OUT), lambda i: (i, 0)),
    out_shape=jax.ShapeDtypeStruct((N_NODES, OUT), jnp.float32),
)


_agg128 = _make_agg(IN_CH, 2, 20)
_agg64 = _make_agg(OUT, 8, 16)


def kernel(x, edge_index, W1, b1, a1, W2, b2, a2):
    pad = EPAD - N_EDGES
    pad_src = jnp.arange(pad, dtype=jnp.int32) % N_NODES
    pad_dst = N_NODES + jnp.arange(pad, dtype=jnp.int32) % (ACC_ROWS - N_NODES)
    ei = jnp.concatenate(
        [edge_index.astype(jnp.int32), jnp.stack([pad_src, pad_dst])], axis=1
    ).reshape(2, NW, C, CHUNK)

    degp = _deg_kernel(ei)
    xp = _prescale(degp, x)
    p = _agg128(xp, ei)
    gp = _mid(p, xp, degp, W1, b1.reshape(1, HID), a1.reshape(1, HID), W2)
    q = _agg64(gp, ei)
    return _final(q, gp, degp, b2.reshape(1, OUT), a2.reshape(1, OUT))
